# baseline (speedup 1.0000x reference)
import functools

import jax
import jax.numpy as jnp
from jax import lax
from jax.experimental import pallas as pl
from jax.experimental.pallas import tpu as pltpu
from jax.experimental.pallas import tpu_sc as plsc

NC, NS, L = 2, 16, 16
CS = 128
T = 64
NBUF = 8
PF = 4


def _make_sc_call(N, D, E):
    TH = N // NC
    n_chunks = TH // T
    n_groups = n_chunks // NBUF

    mesh = plsc.VectorSubcoreMesh(core_axis_name="c", subcore_axis_name="s")

    @functools.partial(
        pl.kernel,
        out_type=jax.ShapeDtypeStruct((N, D), jnp.float32),
        mesh=mesh,
        scratch_types=[
            pltpu.VMEM((E, CS), jnp.float32),
            pltpu.VMEM((TH + L,), jnp.int32),
            pltpu.VMEM((TH + L,), jnp.float32),
            [pltpu.VMEM((T, CS), jnp.float32) for _ in range(NBUF)],
            pltpu.SemaphoreType.DMA((NBUF,)),
            pltpu.SemaphoreType.DMA((NBUF,)),
        ],
    )
    def call(hs_hbm, expert_hbm, routes_hbm, prob_hbm, out_hbm,
             table_v, routes_v, probs_v, bufs, in_sems, out_sems):
        cid = lax.axis_index("c")
        sid = lax.axis_index("s")
        row0 = cid * TH
        col0 = sid * CS

        pltpu.sync_copy(expert_hbm.at[:, pl.ds(col0, CS)], table_v)
        pltpu.sync_copy(routes_hbm.at[pl.ds(row0, TH)],
                        routes_v.at[pl.ds(0, TH)])
        pltpu.sync_copy(prob_hbm.at[pl.ds(row0, TH)],
                        probs_v.at[pl.ds(0, TH)])

        def hbm_block(c):
            return (pl.ds(row0 + c * T, T), pl.ds(col0, CS))

        def issue_in(c, b):
            r, cc = hbm_block(c)
            pltpu.async_copy(hs_hbm.at[r, cc], bufs[b], in_sems.at[b])

        def wait_in(c, b):
            r, cc = hbm_block(c)
            pltpu.make_async_copy(hs_hbm.at[r, cc], bufs[b],
                                  in_sems.at[b]).wait()

        def issue_out(c, b):
            r, cc = hbm_block(c)
            pltpu.async_copy(bufs[b], out_hbm.at[r, cc], out_sems.at[b])

        def wait_out(b):
            r, cc = hbm_block(0)
            pltpu.make_async_copy(bufs[b], out_hbm.at[r, cc],
                                  out_sems.at[b]).wait()

        def compute(c, b):
            hb = bufs[b]
            lbase = c * T

            @plsc.parallel_loop(0, T, 1, unroll=8)
            def tok_body(t):
                lt = lbase + t
                rv = routes_v[pl.ds(lt, L)]
                pvv = probs_v[pl.ds(lt, L)]
                rt = rv[0]
                pv = jnp.broadcast_to(pvv[0], (L,))
                for j in range(CS // L):
                    sl = pl.ds(j * L, L)
                    plsc.addupdate(hb.at[t, sl], table_v[rt, sl] * pv)

        for b in range(PF):
            issue_in(b, b)

        for b in range(NBUF):
            if b >= PF:
                wait_out(b - PF)
            issue_in(b + PF, (b + PF) % NBUF)
            wait_in(b, b)
            compute(b, b)
            issue_out(b, b)

        def group_body(g, carry):
            base = g * NBUF
            for b in range(NBUF):
                c = base + b
                wait_out((b + PF) % NBUF)
                issue_in(c + PF, (b + PF) % NBUF)
                wait_in(c, b)
                compute(c, b)
                issue_out(c, b)
            return carry

        lax.fori_loop(1, n_groups - 1, group_body, 0)

        base = (n_groups - 1) * NBUF
        for b in range(NBUF):
            c = base + b
            if b < PF:
                wait_out((b + PF) % NBUF)
                issue_in(c + PF, (b + PF) % NBUF)
            wait_in(c, b)
            compute(c, b)
            issue_out(c, b)

        for b in range(NBUF):
            wait_out(b)

    return call


def kernel(hidden_states, expert_output, routes, route_prob_max):
    b, s, d = hidden_states.shape
    e = expert_output.shape[0]
    n = b * s
    hs2 = hidden_states.reshape(n, d)
    routes_i32 = routes.astype(jnp.int32)
    out = _make_sc_call(n, d, e)(
        hs2, expert_output, routes_i32, route_prob_max)
    return out.reshape(b, s, d)


# speedup vs baseline: 1.4587x; 1.4587x over previous
import functools

import jax
import jax.numpy as jnp
from jax import lax
from jax.experimental import pallas as pl
from jax.experimental.pallas import tpu as pltpu
from jax.experimental.pallas import tpu_sc as plsc

NC, NS, L = 2, 16, 16
CS = 128
T = 64
NBUF = 8
PF = 4


def _make_sc_call(N, D, E):
    TH = N // NC
    n_chunks = TH // T
    n_groups = n_chunks // NBUF

    mesh = plsc.VectorSubcoreMesh(core_axis_name="c", subcore_axis_name="s")

    @functools.partial(
        pl.kernel,
        out_type=jax.ShapeDtypeStruct((N, D), jnp.float32),
        mesh=mesh,
        scratch_types=[
            pltpu.VMEM((E, CS), jnp.float32),
            pltpu.VMEM((TH + L,), jnp.int32),
            pltpu.VMEM((TH + L,), jnp.float32),
            [pltpu.VMEM((T, CS), jnp.float32) for _ in range(NBUF)],
            pltpu.SemaphoreType.DMA((NBUF,)),
            pltpu.SemaphoreType.DMA((NBUF,)),
        ],
    )
    def call(hs_hbm, expert_hbm, routes_hbm, prob_hbm, out_hbm,
             table_v, routes_v, probs_v, bufs, in_sems, out_sems):
        cid = lax.axis_index("c")
        sid = lax.axis_index("s")
        row0 = cid * TH
        col0 = sid * CS

        pltpu.sync_copy(expert_hbm.at[:, pl.ds(col0, CS)], table_v)
        pltpu.sync_copy(routes_hbm.at[pl.ds(row0, TH)],
                        routes_v.at[pl.ds(0, TH)])
        pltpu.sync_copy(prob_hbm.at[pl.ds(row0, TH)],
                        probs_v.at[pl.ds(0, TH)])

        def hbm_block(c):
            return (pl.ds(row0 + c * T, T), pl.ds(col0, CS))

        def issue_in(c, b):
            r, cc = hbm_block(c)
            pltpu.async_copy(hs_hbm.at[r, cc], bufs[b], in_sems.at[b])

        def wait_in(c, b):
            r, cc = hbm_block(c)
            pltpu.make_async_copy(hs_hbm.at[r, cc], bufs[b],
                                  in_sems.at[b]).wait()

        def issue_out(c, b):
            r, cc = hbm_block(c)
            pltpu.async_copy(bufs[b], out_hbm.at[r, cc], out_sems.at[b])

        def wait_out(b):
            r, cc = hbm_block(0)
            pltpu.make_async_copy(bufs[b], out_hbm.at[r, cc],
                                  out_sems.at[b]).wait()

        def compute(c, b):
            hb = bufs[b]
            lbase = c * T

            @plsc.parallel_loop(0, T, 1, unroll=4)
            def tok_body(t):
                lt = lbase + t
                rv = routes_v[pl.ds(lt, L)]
                pvv = probs_v[pl.ds(lt, L)]
                rt = rv[0]
                pv = jnp.broadcast_to(pvv[0], (L,))
                for j in range(CS // L):
                    sl = pl.ds(j * L, L)
                    plsc.addupdate(hb.at[t, sl], table_v[rt, sl] * pv)

        for b in range(PF):
            issue_in(b, b)

        for b in range(NBUF):
            if b >= PF:
                wait_out(b - PF)
            issue_in(b + PF, (b + PF) % NBUF)
            wait_in(b, b)
            compute(b, b)
            issue_out(b, b)

        def group_body(g, carry):
            base = g * NBUF
            for b in range(NBUF):
                c = base + b
                wait_out((b + PF) % NBUF)
                issue_in(c + PF, (b + PF) % NBUF)
                wait_in(c, b)
                compute(c, b)
                issue_out(c, b)
            return carry

        lax.fori_loop(1, n_groups - 1, group_body, 0)

        base = (n_groups - 1) * NBUF
        for b in range(NBUF):
            c = base + b
            if b < PF:
                wait_out((b + PF) % NBUF)
                issue_in(c + PF, (b + PF) % NBUF)
            wait_in(c, b)
            compute(c, b)
            issue_out(c, b)

        for b in range(NBUF):
            wait_out(b)

    return call


def _make_tc_call(N, D, E, TB):
    def body(r_ref, p_ref, hid_ref, exp_ref, out_ref):
        r = r_ref[...]
        oh = (r == lax.broadcasted_iota(jnp.int32, (TB, E), 1)
              ).astype(jnp.float32)
        rows = jnp.dot(oh, exp_ref[...], preferred_element_type=jnp.float32)
        out_ref[...] = hid_ref[...] + rows * p_ref[...]

    return pl.pallas_call(
        body,
        grid=(N // TB,),
        in_specs=[
            pl.BlockSpec((TB, 1), lambda i: (i, 0)),
            pl.BlockSpec((TB, 1), lambda i: (i, 0)),
            pl.BlockSpec((TB, D), lambda i: (i, 0)),
            pl.BlockSpec((E, D), lambda i: (0, 0)),
        ],
        out_specs=pl.BlockSpec((TB, D), lambda i: (i, 0)),
        out_shape=jax.ShapeDtypeStruct((N, D), jnp.float32),
    )


def kernel(hidden_states, expert_output, routes, route_prob_max):
    b, s, d = hidden_states.shape
    e = expert_output.shape[0]
    n = b * s
    hs2 = hidden_states.reshape(n, d)
    routes_i32 = routes.astype(jnp.int32)
    out = _make_tc_call(n, d, e, TB=512)(
        routes_i32[:, None], route_prob_max[:, None], hs2, expert_output)
    return out.reshape(b, s, d)
